# baseline (device time: 14424 ns/iter reference)
import jax
import jax.numpy as jnp
from jax import lax
from jax.experimental import pallas as pl
from jax.experimental.pallas import tpu as pltpu

N_DEV = 8
_NHALF = 2


def kernel(x):
    m, n = x.shape
    piece = m // N_DEV
    half = piece // _NHALF

    def body(x_ref, out_ref, rs_ref, rs_send, rs_recv, ag_send, ag_recv):
        my = lax.axis_index("i")

        barrier_sem = pltpu.get_barrier_semaphore()
        for r in range(1, N_DEV):
            pl.semaphore_signal(
                barrier_sem, inc=1,
                device_id=(my ^ r,), device_id_type=pl.DeviceIdType.MESH,
            )
        pl.semaphore_wait(barrier_sem, N_DEV - 1)

        rs = {}
        for hf in range(_NHALF):
            for r in range(1, N_DEV):
                tgt = my ^ r
                rdma = pltpu.make_async_remote_copy(
                    src_ref=x_ref.at[pl.ds(tgt * piece + hf * half, half), :],
                    dst_ref=rs_ref.at[hf, r],
                    send_sem=rs_send.at[hf, r],
                    recv_sem=rs_recv.at[hf, r],
                    device_id=(tgt,),
                    device_id_type=pl.DeviceIdType.MESH,
                )
                rdma.start()
                rs[hf, r] = rdma

        ag = {}
        for hf in range(_NHALF):
            acc = x_ref[pl.ds(my * piece + hf * half, half), :]
            for r in range(1, N_DEV):
                rs[hf, r].wait()
                acc = acc + rs_ref[hf, r]
            out_ref[pl.ds(my * piece + hf * half, half), :] = acc
            for r in range(1, N_DEV):
                tgt = my ^ r
                rdma = pltpu.make_async_remote_copy(
                    src_ref=out_ref.at[pl.ds(my * piece + hf * half, half), :],
                    dst_ref=out_ref.at[pl.ds(my * piece + hf * half, half), :],
                    send_sem=ag_send.at[hf, r],
                    recv_sem=ag_recv.at[hf, r],
                    device_id=(tgt,),
                    device_id_type=pl.DeviceIdType.MESH,
                )
                rdma.start()
                ag[hf, r] = rdma
        for hf in range(_NHALF):
            for r in range(1, N_DEV):
                ag[hf, r].wait()

    return pl.pallas_call(
        body,
        out_shape=jax.ShapeDtypeStruct((m, n), x.dtype),
        in_specs=[pl.BlockSpec(memory_space=pltpu.VMEM)],
        out_specs=pl.BlockSpec(memory_space=pltpu.VMEM),
        scratch_shapes=[
            pltpu.VMEM((_NHALF, N_DEV, half, n), x.dtype),
            pltpu.SemaphoreType.DMA((_NHALF, N_DEV)),
            pltpu.SemaphoreType.DMA((_NHALF, N_DEV)),
            pltpu.SemaphoreType.DMA((_NHALF, N_DEV)),
            pltpu.SemaphoreType.DMA((_NHALF, N_DEV)),
        ],
        compiler_params=pltpu.CompilerParams(collective_id=0),
    )(x)


# device time: 13317 ns/iter; 1.0831x vs baseline; 1.0831x over previous
import jax
from jax import lax
from jax.experimental import pallas as pl
from jax.experimental.pallas import tpu as pltpu

N_DEV = 8
_SEND_ORDER = (6, 2, 5, 7, 3, 4, 1)
_WAIT_ORDER = (1, 3, 4, 2, 5, 7, 6)


def kernel(x):
    m, n = x.shape
    piece = m // N_DEV

    def body(x_ref, out_ref, rs_ref, rs_send, rs_recv, ag_send, ag_recv):
        my = lax.axis_index("i")

        barrier_sem = pltpu.get_barrier_semaphore()
        for r in range(1, N_DEV):
            pl.semaphore_signal(
                barrier_sem, inc=1,
                device_id=(my ^ r,), device_id_type=pl.DeviceIdType.MESH,
            )
        pl.semaphore_wait(barrier_sem, N_DEV - 1)

        rs = {}
        for r in _SEND_ORDER:
            tgt = my ^ r
            rs[r] = pltpu.make_async_remote_copy(
                src_ref=x_ref.at[pl.ds(tgt * piece, piece), :],
                dst_ref=rs_ref.at[r],
                send_sem=rs_send.at[r],
                recv_sem=rs_recv.at[r],
                device_id=(tgt,),
                device_id_type=pl.DeviceIdType.MESH,
            )
            rs[r].start()

        acc = x_ref[pl.ds(my * piece, piece), :]
        for r in _WAIT_ORDER:
            rs[r].wait()
            acc = acc + rs_ref[r]
        out_ref[pl.ds(my * piece, piece), :] = acc

        ag = {}
        for r in _SEND_ORDER:
            tgt = my ^ r
            ag[r] = pltpu.make_async_remote_copy(
                src_ref=out_ref.at[pl.ds(my * piece, piece), :],
                dst_ref=out_ref.at[pl.ds(my * piece, piece), :],
                send_sem=ag_send.at[r],
                recv_sem=ag_recv.at[r],
                device_id=(tgt,),
                device_id_type=pl.DeviceIdType.MESH,
            )
            ag[r].start()
        for r in _WAIT_ORDER:
            ag[r].wait()

    return pl.pallas_call(
        body,
        out_shape=jax.ShapeDtypeStruct((m, n), x.dtype),
        in_specs=[pl.BlockSpec(memory_space=pltpu.VMEM)],
        out_specs=pl.BlockSpec(memory_space=pltpu.VMEM),
        scratch_shapes=[
            pltpu.VMEM((N_DEV, piece, n), x.dtype),
            pltpu.SemaphoreType.DMA((N_DEV,)),
            pltpu.SemaphoreType.DMA((N_DEV,)),
            pltpu.SemaphoreType.DMA((N_DEV,)),
            pltpu.SemaphoreType.DMA((N_DEV,)),
        ],
        compiler_params=pltpu.CompilerParams(collective_id=0),
    )(x)
